# baseline (speedup 1.0000x reference)
import functools

import jax
import jax.numpy as jnp
from jax import lax
from jax.experimental import pallas as pl
from jax.experimental.pallas import tpu as pltpu
from jax.experimental.pallas import tpu_sc as plsc

N_BUS, N_LOAD, N_LINE, N_GEN = 20000, 15000, 9000, 6000
N_NODES = N_BUS + N_LOAD + N_LINE + N_GEN
N_EDGES = 800000
EMBED = 64
NUM_GRAPHS = N_GEN // 6
GEN_START = N_NODES - N_GEN

NC, NS = 2, 16
NW = NC * NS
EPW = N_EDGES // NW
CHUNK = 5000
NITER = (CHUNK + 15) // 16
K = 1024
ROWS_PER_SUB = 376
ACC_ROWS = ROWS_PER_SUB * NS
DUMMY = N_GEN


def _sc_segsum_body(src_hbm, dst_hbm, h_hbm, acc_out, cnt_out,
                    msrc, mdst, sidx, didx, rows, ones_buf,
                    acc_sh, cnt_sh, sem):
    cid = lax.axis_index("c")
    sid = lax.axis_index("s")
    wid = sid * NC + cid

    zf = jnp.zeros((16,), jnp.float32)
    zi = jnp.zeros((16,), jnp.int32)
    ones16 = jnp.ones((16,), jnp.float32)
    lanes = lax.iota(jnp.int32, 16)

    def _zrows(i, _):
        rows[i, pl.ds(0, 16)] = zf
        rows[i, pl.ds(16, 16)] = zf
        rows[i, pl.ds(32, 16)] = zf
        rows[i, pl.ds(48, 16)] = zf
        return 0
    lax.fori_loop(0, ROWS_PER_SUB, _zrows, 0)

    def _fill_z16(i, _):
        ones_buf[i, pl.ds(0, 16)] = zf
        return 0
    lax.fori_loop(0, K, _fill_z16, 0)

    pltpu.sync_copy(rows.at[pl.ds(0, ROWS_PER_SUB)],
                    acc_sh.at[pl.ds(sid * ROWS_PER_SUB, ROWS_PER_SUB)])
    pltpu.sync_copy(ones_buf.at[pl.ds(0, ROWS_PER_SUB)],
                    cnt_sh.at[pl.ds(sid * ROWS_PER_SUB, ROWS_PER_SUB)])

    one_hot = jnp.where(lanes < 1, 1.0, 0.0).astype(jnp.float32)

    def _fill_ones(i, _):
        ones_buf[i, pl.ds(0, 16)] = one_hot
        return 0
    lax.fori_loop(0, K, _fill_ones, 0)
    plsc.subcore_barrier()

    ebase = wid * EPW
    for k in range(EPW // CHUNK):
        base = ebase + k * CHUNK
        pltpu.sync_copy(src_hbm.at[pl.ds(base, CHUNK)],
                        msrc.at[pl.ds(0, CHUNK)])
        pltpu.sync_copy(dst_hbm.at[pl.ds(base, CHUNK)],
                        mdst.at[pl.ds(0, CHUNK)])

        def _compact(i, cursor):
            s = msrc[pl.ds(i * 16, 16)]
            d = mdst[pl.ds(i * 16, 16)]
            valid = (i * 16 + lanes) < CHUNK
            m = jnp.logical_and(d >= GEN_START, valid)
            dl = d - GEN_START
            plsc.store_compressed(msrc.at[pl.ds(cursor, 16)], s, mask=m)
            plsc.store_compressed(mdst.at[pl.ds(cursor, 16)], dl, mask=m)
            return cursor + jnp.sum(m.astype(jnp.int32))

        num_match = lax.fori_loop(0, NITER, _compact, jnp.int32(0))

        dummy16 = jnp.full((16,), DUMMY, jnp.int32)

        def _pad(j, _):
            msrc[pl.ds(num_match + j * 16, 16)] = zi
            mdst[pl.ds(num_match + j * 16, 16)] = dummy16
            return 0
        lax.fori_loop(0, K // 16, _pad, 0)

        nsub = (num_match + (K - 1)) // K

        def _round2(j, _):
            def _stage(t, _c):
                sidx[pl.ds(t * 16, 16)] = msrc[pl.ds(j * K + t * 16, 16)]
                didx[pl.ds(t * 16, 16)] = mdst[pl.ds(j * K + t * 16, 16)]
                return 0
            lax.fori_loop(0, K // 16, _stage, 0)
            pltpu.async_copy(h_hbm.at[sidx], rows, sem).wait()
            pltpu.sync_copy(rows, acc_sh.at[didx], add=True)
            pltpu.sync_copy(ones_buf, cnt_sh.at[didx], add=True)
            return 0
        lax.fori_loop(0, nsub, _round2, 0)

    plsc.subcore_barrier()
    pltpu.sync_copy(acc_sh.at[pl.ds(sid * ROWS_PER_SUB, ROWS_PER_SUB)],
                    acc_out.at[cid, pl.ds(sid * ROWS_PER_SUB, ROWS_PER_SUB)])
    pltpu.sync_copy(cnt_sh.at[pl.ds(sid * ROWS_PER_SUB, ROWS_PER_SUB)],
                    cnt_out.at[cid, pl.ds(sid * ROWS_PER_SUB, ROWS_PER_SUB)])


@functools.cache
def _sc_segsum():
    mesh = plsc.VectorSubcoreMesh(
        core_axis_name="c", subcore_axis_name="s",
        num_cores=NC, num_subcores=NS)
    return pl.kernel(
        _sc_segsum_body,
        out_type=(
            jax.ShapeDtypeStruct((NC, ACC_ROWS, EMBED), jnp.float32),
            jax.ShapeDtypeStruct((NC, ACC_ROWS, 16), jnp.float32),
        ),
        mesh=mesh,
        compiler_params=pltpu.CompilerParams(
            needs_layout_passes=False, use_tc_tiling_on_sc=False),
        scratch_types=[
            pltpu.VMEM((CHUNK + K + 32,), jnp.int32),
            pltpu.VMEM((CHUNK + K + 32,), jnp.int32),
            pltpu.VMEM((K,), jnp.int32),
            pltpu.VMEM((K,), jnp.int32),
            pltpu.VMEM((K, EMBED), jnp.float32),
            pltpu.VMEM((K, 16), jnp.float32),
            pltpu.VMEM_SHARED((ACC_ROWS, EMBED), jnp.float32),
            pltpu.VMEM_SHARED((ACC_ROWS, 16), jnp.float32),
            pltpu.SemaphoreType.DMA,
        ],
    )


EMB_BLK = 1000
FPAD = 16


def _embed_body(x_ref, w_ref, b_ref, o_ref):
    o_ref[...] = (
        jnp.dot(x_ref[...], w_ref[0], preferred_element_type=jnp.float32)
        + b_ref[0])


def _type_of_block(i):
    i = i.astype(jnp.int32) if hasattr(i, "astype") else jnp.int32(i)
    return ((i >= N_BUS // EMB_BLK).astype(jnp.int32)
            + (i >= (N_BUS + N_LOAD) // EMB_BLK).astype(jnp.int32)
            + (i >= GEN_START // EMB_BLK).astype(jnp.int32))


_embed = pl.pallas_call(
    _embed_body,
    grid=(N_NODES // EMB_BLK,),
    in_specs=[
        pl.BlockSpec((EMB_BLK, FPAD), lambda i: (i, 0)),
        pl.BlockSpec((1, FPAD, EMBED), lambda i: (_type_of_block(i), 0, 0)),
        pl.BlockSpec((1, 1, EMBED), lambda i: (_type_of_block(i), 0, 0)),
    ],
    out_specs=pl.BlockSpec((EMB_BLK, EMBED), lambda i: (i, 0)),
    out_shape=jax.ShapeDtypeStruct((N_NODES, EMBED), jnp.float32),
)


def _final_body(h_ref, acc_ref, cnt_ref, wc_ref, bc_ref, wf_ref, bf_ref, o_ref):
    hg = h_ref[...]
    av = acc_ref[...]
    a = av[0, :N_GEN] + av[1, :N_GEN]
    cv = cnt_ref[...]
    c = cv[0, :N_GEN, 0] + cv[1, :N_GEN, 0]
    wc = wc_ref[...]
    wa = wc[:, :EMBED]
    wb = wc[:, EMBED:]
    dot_a = lax.dot_general(hg, wa - wb, (((1,), (1,)), ((), ())),
                            preferred_element_type=jnp.float32)
    dot_b = lax.dot_general(a, wb, (((1,), (1,)), ((), ())),
                            preferred_element_type=jnp.float32)
    numer = c[:, None] * (dot_a + bc_ref[...]) + dot_b
    h2 = jnp.maximum(numer / jnp.maximum(c[:, None], 1.0), 0.0)
    wf = wf_ref[...]
    out = (jnp.sum(h2 * wf[:, :EMBED], axis=1)
           + jnp.sum(hg * wf[:, EMBED:], axis=1))
    o_ref[...] = out[:, None] + bf_ref[...]


_final = pl.pallas_call(
    _final_body,
    out_shape=jax.ShapeDtypeStruct((N_GEN, 1), jnp.float32),
)


def kernel(x_bus, x_load, x_line, x_gen, edge_index,
           W_bus, b_bus, W_load, b_load, W_line, b_line, W_gen, b_gen,
           W_conv, b_conv, W_final, b_final, W_val, b_val):
    def padw(w):
        return jnp.pad(w.T, ((0, FPAD - w.shape[1]), (0, 0)))

    X = jnp.concatenate([
        jnp.pad(x_bus, ((0, 0), (0, FPAD - x_bus.shape[1]))),
        jnp.pad(x_load, ((0, 0), (0, FPAD - x_load.shape[1]))),
        jnp.pad(x_line, ((0, 0), (0, FPAD - x_line.shape[1]))),
        jnp.pad(x_gen, ((0, 0), (0, FPAD - x_gen.shape[1]))),
    ], axis=0)
    Wstack = jnp.stack([padw(W_bus), padw(W_load), padw(W_line), padw(W_gen)])
    bstack = jnp.stack([b_bus, b_load, b_line, b_gen]).reshape(4, 1, EMBED)

    h = _embed(X, Wstack, bstack)
    ei = edge_index.astype(jnp.int32)
    acc, cnt = _sc_segsum()(ei[0], ei[1], h)
    out = _final(h[GEN_START:], acc, cnt, W_conv,
                 b_conv.reshape(1, EMBED), W_final, b_final.reshape(1, 1))
    return out.reshape(NUM_GRAPHS, N_GEN // NUM_GRAPHS)


# speedup vs baseline: 4.4706x; 4.4706x over previous
import functools

import jax
import jax.numpy as jnp
from jax import lax
from jax.experimental import pallas as pl
from jax.experimental.pallas import tpu as pltpu
from jax.experimental.pallas import tpu_sc as plsc

N_BUS, N_LOAD, N_LINE, N_GEN = 20000, 15000, 9000, 6000
N_NODES = N_BUS + N_LOAD + N_LINE + N_GEN
N_EDGES = 800000
EMBED = 64
NUM_GRAPHS = N_GEN // 6
GEN_START = N_NODES - N_GEN

NC, NS = 2, 16
NW = NC * NS
EPW = N_EDGES // NW
CHUNK = 5000
NITER = (CHUNK + 15) // 16
K = 128
NBUF = 8
ROWS_PER_SUB = 376
ACC_ROWS = ROWS_PER_SUB * NS
DUMMY = N_GEN


def _sc_segsum_body(src_hbm, dst_hbm, h_hbm, acc_out, cnt_out,
                    msrc, mdst, sidx, didx, rows, ones_buf,
                    acc_sh, cnt_sh, *sems):
    cid = lax.axis_index("c")
    sid = lax.axis_index("s")
    wid = sid * NC + cid

    zf = jnp.zeros((16,), jnp.float32)
    zi = jnp.zeros((16,), jnp.int32)
    lanes = lax.iota(jnp.int32, 16)

    def _zrows(i, _):
        rows[0, i, pl.ds(0, 16)] = zf
        rows[0, i, pl.ds(16, 16)] = zf
        rows[0, i, pl.ds(32, 16)] = zf
        rows[0, i, pl.ds(48, 16)] = zf
        return 0
    lax.fori_loop(0, K, _zrows, 0)

    def _fill_z16(i, _):
        ones_buf[i, pl.ds(0, 16)] = zf
        return 0
    lax.fori_loop(0, K, _fill_z16, 0)

    abase = sid * ROWS_PER_SUB
    pltpu.sync_copy(rows.at[0], acc_sh.at[pl.ds(abase, 128)])
    pltpu.sync_copy(rows.at[0], acc_sh.at[pl.ds(abase + 128, 128)])
    pltpu.sync_copy(rows.at[0, pl.ds(0, 120)],
                    acc_sh.at[pl.ds(abase + 256, 120)])
    pltpu.sync_copy(ones_buf, cnt_sh.at[pl.ds(abase, 128)])
    pltpu.sync_copy(ones_buf, cnt_sh.at[pl.ds(abase + 128, 128)])
    pltpu.sync_copy(ones_buf.at[pl.ds(0, 120)],
                    cnt_sh.at[pl.ds(abase + 256, 120)])

    one_hot = jnp.where(lanes < 1, 1.0, 0.0).astype(jnp.float32)

    def _fill_ones(i, _):
        ones_buf[i, pl.ds(0, 16)] = one_hot
        return 0
    lax.fori_loop(0, K, _fill_ones, 0)
    plsc.subcore_barrier()

    ebase = wid * EPW
    for k in range(EPW // CHUNK):
        base = ebase + k * CHUNK
        pltpu.sync_copy(src_hbm.at[pl.ds(base, CHUNK)],
                        msrc.at[pl.ds(0, CHUNK)])
        pltpu.sync_copy(dst_hbm.at[pl.ds(base, CHUNK)],
                        mdst.at[pl.ds(0, CHUNK)])

        def _compact(i, cursor):
            s = msrc[pl.ds(i * 16, 16)]
            d = mdst[pl.ds(i * 16, 16)]
            valid = (i * 16 + lanes) < CHUNK
            m = jnp.logical_and(d >= GEN_START, valid)
            dl = d - GEN_START
            plsc.store_compressed(msrc.at[pl.ds(cursor, 16)], s, mask=m)
            plsc.store_compressed(mdst.at[pl.ds(cursor, 16)], dl, mask=m)
            return cursor + jnp.sum(m.astype(jnp.int32))

        num_match = lax.fori_loop(0, NITER, _compact, jnp.int32(0))

        dummy16 = jnp.full((16,), DUMMY, jnp.int32)

        def _pad(j, _):
            msrc[pl.ds(num_match + j * 16, 16)] = zi
            mdst[pl.ds(num_match + j * 16, 16)] = dummy16
            return 0
        lax.fori_loop(0, K // 16, _pad, 0)

        nsub = (num_match + (K - 1)) // K
        ngroups = (nsub + (NBUF - 1)) // NBUF

        def _group(g, _):
            jbase = g * NBUF
            for b in range(NBUF):
                j = jbase + b

                @pl.when(j < nsub)
                def _fire():
                    def _stage(t, _c):
                        sidx[b, pl.ds(t * 16, 16)] = (
                            msrc[pl.ds(j * K + t * 16, 16)])
                        didx[b, pl.ds(t * 16, 16)] = (
                            mdst[pl.ds(j * K + t * 16, 16)])
                        return 0
                    lax.fori_loop(0, K // 16, _stage, 0)
                    pltpu.async_copy(h_hbm.at[sidx.at[b]], rows.at[b],
                                     sems[b])
            for b in range(NBUF):
                j = jbase + b

                @pl.when(j < nsub)
                def _drain():
                    pltpu.make_async_copy(h_hbm.at[sidx.at[b]], rows.at[b],
                                          sems[b]).wait()
                    pltpu.sync_copy(rows.at[b], acc_sh.at[didx.at[b]],
                                    add=True)
                    pltpu.sync_copy(ones_buf, cnt_sh.at[didx.at[b]],
                                    add=True)
            return 0
        lax.fori_loop(0, ngroups, _group, 0)

    plsc.subcore_barrier()
    pltpu.sync_copy(acc_sh.at[pl.ds(sid * ROWS_PER_SUB, ROWS_PER_SUB)],
                    acc_out.at[cid, pl.ds(sid * ROWS_PER_SUB, ROWS_PER_SUB)])
    pltpu.sync_copy(cnt_sh.at[pl.ds(sid * ROWS_PER_SUB, ROWS_PER_SUB)],
                    cnt_out.at[cid, pl.ds(sid * ROWS_PER_SUB, ROWS_PER_SUB)])


@functools.cache
def _sc_segsum():
    mesh = plsc.VectorSubcoreMesh(
        core_axis_name="c", subcore_axis_name="s",
        num_cores=NC, num_subcores=NS)
    return pl.kernel(
        _sc_segsum_body,
        out_type=(
            jax.ShapeDtypeStruct((NC, ACC_ROWS, EMBED), jnp.float32),
            jax.ShapeDtypeStruct((NC, ACC_ROWS, 16), jnp.float32),
        ),
        mesh=mesh,
        compiler_params=pltpu.CompilerParams(
            needs_layout_passes=False, use_tc_tiling_on_sc=False),
        scratch_types=[
            pltpu.VMEM((CHUNK + K + 32,), jnp.int32),
            pltpu.VMEM((CHUNK + K + 32,), jnp.int32),
            pltpu.VMEM((NBUF, K), jnp.int32),
            pltpu.VMEM((NBUF, K), jnp.int32),
            pltpu.VMEM((NBUF, K, EMBED), jnp.float32),
            pltpu.VMEM((K, 16), jnp.float32),
            pltpu.VMEM_SHARED((ACC_ROWS, EMBED), jnp.float32),
            pltpu.VMEM_SHARED((ACC_ROWS, 16), jnp.float32),
        ] + [pltpu.SemaphoreType.DMA] * NBUF,
    )


EMB_BLK = 1000
FPAD = 16


def _embed_body(x_ref, w_ref, b_ref, o_ref):
    o_ref[...] = (
        jnp.dot(x_ref[...], w_ref[0], preferred_element_type=jnp.float32)
        + b_ref[0])


def _type_of_block(i):
    i = i.astype(jnp.int32) if hasattr(i, "astype") else jnp.int32(i)
    return ((i >= N_BUS // EMB_BLK).astype(jnp.int32)
            + (i >= (N_BUS + N_LOAD) // EMB_BLK).astype(jnp.int32)
            + (i >= GEN_START // EMB_BLK).astype(jnp.int32))


_embed = pl.pallas_call(
    _embed_body,
    grid=(N_NODES // EMB_BLK,),
    in_specs=[
        pl.BlockSpec((EMB_BLK, FPAD), lambda i: (i, 0)),
        pl.BlockSpec((1, FPAD, EMBED), lambda i: (_type_of_block(i), 0, 0)),
        pl.BlockSpec((1, 1, EMBED), lambda i: (_type_of_block(i), 0, 0)),
    ],
    out_specs=pl.BlockSpec((EMB_BLK, EMBED), lambda i: (i, 0)),
    out_shape=jax.ShapeDtypeStruct((N_NODES, EMBED), jnp.float32),
)


def _final_body(h_ref, acc_ref, cnt_ref, wc_ref, bc_ref, wf_ref, bf_ref, o_ref):
    hg = h_ref[...]
    av = acc_ref[...]
    a = av[0, :N_GEN] + av[1, :N_GEN]
    cv = cnt_ref[...]
    c = cv[0, :N_GEN, 0] + cv[1, :N_GEN, 0]
    wc = wc_ref[...]
    wa = wc[:, :EMBED]
    wb = wc[:, EMBED:]
    dot_a = lax.dot_general(hg, wa - wb, (((1,), (1,)), ((), ())),
                            preferred_element_type=jnp.float32)
    dot_b = lax.dot_general(a, wb, (((1,), (1,)), ((), ())),
                            preferred_element_type=jnp.float32)
    numer = c[:, None] * (dot_a + bc_ref[...]) + dot_b
    h2 = jnp.maximum(numer / jnp.maximum(c[:, None], 1.0), 0.0)
    wf = wf_ref[...]
    out = (jnp.sum(h2 * wf[:, :EMBED], axis=1)
           + jnp.sum(hg * wf[:, EMBED:], axis=1))
    o_ref[...] = out[:, None] + bf_ref[...]


_final = pl.pallas_call(
    _final_body,
    out_shape=jax.ShapeDtypeStruct((N_GEN, 1), jnp.float32),
)


def kernel(x_bus, x_load, x_line, x_gen, edge_index,
           W_bus, b_bus, W_load, b_load, W_line, b_line, W_gen, b_gen,
           W_conv, b_conv, W_final, b_final, W_val, b_val):
    def padw(w):
        return jnp.pad(w.T, ((0, FPAD - w.shape[1]), (0, 0)))

    X = jnp.concatenate([
        jnp.pad(x_bus, ((0, 0), (0, FPAD - x_bus.shape[1]))),
        jnp.pad(x_load, ((0, 0), (0, FPAD - x_load.shape[1]))),
        jnp.pad(x_line, ((0, 0), (0, FPAD - x_line.shape[1]))),
        jnp.pad(x_gen, ((0, 0), (0, FPAD - x_gen.shape[1]))),
    ], axis=0)
    Wstack = jnp.stack([padw(W_bus), padw(W_load), padw(W_line), padw(W_gen)])
    bstack = jnp.stack([b_bus, b_load, b_line, b_gen]).reshape(4, 1, EMBED)

    h = _embed(X, Wstack, bstack)
    ei = edge_index.astype(jnp.int32)
    acc, cnt = _sc_segsum()(ei[0], ei[1], h)
    out = _final(h[GEN_START:], acc, cnt, W_conv,
                 b_conv.reshape(1, EMBED), W_final, b_final.reshape(1, 1))
    return out.reshape(NUM_GRAPHS, N_GEN // NUM_GRAPHS)
